# baseline (device time: 24172 ns/iter reference)
import jax
import jax.numpy as jnp
from jax import lax
from jax.experimental import pallas as pl
from jax.experimental.pallas import tpu as pltpu

P = 16
HALVES = 2


def kernel(x, w_mat):
    m_per, k = x.shape
    _, n = w_mat.shape
    n_per = n // P
    m = m_per * P
    n_half = n // HALVES
    blocks_per_half = P // HALVES

    def body(x_hbm, w_hbm, out_hbm, x_vmem, w_vmem, y_bf, recv_buf, out_f32,
             in_sems, send_sems, recv_sems, ready_sems):
        my = lax.axis_index("i")

        x_cp = pltpu.make_async_copy(x_hbm, x_vmem, in_sems.at[0])
        x_cp.start()
        w_cps = []
        for h in range(HALVES):
            cp = pltpu.make_async_copy(
                w_hbm.at[:, pl.ds(h * n_half, n_half)],
                w_vmem.at[:, pl.ds(h * n_half, n_half)],
                in_sems.at[1 + h],
            )
            cp.start()
            w_cps.append(cp)

        for d in range(1, P):
            pl.semaphore_signal(
                ready_sems.at[my], inc=1,
                device_id=((my + d) % P,),
                device_id_type=pl.DeviceIdType.MESH,
            )

        barrier_sem = pltpu.get_barrier_semaphore()
        for nbr in [(my + 1) % P, (my - 1) % P]:
            pl.semaphore_signal(
                barrier_sem, inc=1,
                device_id=(nbr,), device_id_type=pl.DeviceIdType.MESH,
            )
        pl.semaphore_wait(barrier_sem, 2)

        x_cp.wait()
        for h in range(HALVES):
            w_cps[h].wait()
            y_half = jnp.dot(
                x_vmem[...],
                w_vmem[:, pl.ds(h * n_half, n_half)],
                preferred_element_type=jnp.float32,
            ).astype(jnp.bfloat16)
            y_bf[:, pl.ds(h * n_half, n_half)] = y_half
            for c in range(h * blocks_per_half, (h + 1) * blocks_per_half):

                @pl.when(c == my)
                def _own():
                    recv_buf[pl.ds(my * m_per, m_per), :] = y_bf[
                        :, pl.ds(my * n_per, n_per)
                    ]

                @pl.when(c != my)
                def _send():
                    pl.semaphore_wait(ready_sems.at[c], 1)
                    rdma = pltpu.make_async_remote_copy(
                        src_ref=y_bf.at[:, pl.ds(c * n_per, n_per)],
                        dst_ref=recv_buf.at[pl.ds(my * m_per, m_per), :],
                        send_sem=send_sems.at[c],
                        recv_sem=recv_sems.at[my],
                        device_id=(c,),
                        device_id_type=pl.DeviceIdType.MESH,
                    )
                    rdma.start()

        out_f32[pl.ds(my * m_per, m_per), :] = recv_buf[
            pl.ds(my * m_per, m_per), :
        ].astype(jnp.float32)

        for d in range(1, P):
            src = (my - d) % P
            recv = pltpu.make_async_remote_copy(
                src_ref=y_bf.at[:, pl.ds(0, n_per)],
                dst_ref=recv_buf.at[pl.ds(src * m_per, m_per), :],
                send_sem=send_sems.at[0],
                recv_sem=recv_sems.at[src],
                device_id=(my,),
                device_id_type=pl.DeviceIdType.MESH,
            )
            recv.wait_recv()
            out_f32[pl.ds(src * m_per, m_per), :] = recv_buf[
                pl.ds(src * m_per, m_per), :
            ].astype(jnp.float32)

        out_cp = pltpu.make_async_copy(out_f32, out_hbm, in_sems.at[3])
        out_cp.start()

        for c in range(P):

            @pl.when(c != my)
            def _drain():
                send = pltpu.make_async_remote_copy(
                    src_ref=y_bf.at[:, pl.ds(0, n_per)],
                    dst_ref=recv_buf.at[pl.ds(0, m_per), :],
                    send_sem=send_sems.at[c],
                    recv_sem=recv_sems.at[my],
                    device_id=(my,),
                    device_id_type=pl.DeviceIdType.MESH,
                )
                send.wait_send()

        out_cp.wait()

    hbm = pl.BlockSpec(memory_space=pltpu.MemorySpace.HBM)
    return pl.pallas_call(
        body,
        out_shape=jax.ShapeDtypeStruct((m, n_per), jnp.float32),
        in_specs=[hbm, hbm],
        out_specs=hbm,
        scratch_shapes=[
            pltpu.VMEM((m_per, k), jnp.float32),
            pltpu.VMEM((k, n), jnp.float32),
            pltpu.VMEM((m_per, n), jnp.bfloat16),
            pltpu.VMEM((m, n_per), jnp.bfloat16),
            pltpu.VMEM((m, n_per), jnp.float32),
            pltpu.SemaphoreType.DMA((4,)),
            pltpu.SemaphoreType.DMA((P,)),
            pltpu.SemaphoreType.DMA((P,)),
            pltpu.SemaphoreType.REGULAR((P,)),
        ],
        compiler_params=pltpu.CompilerParams(collective_id=0),
    )(x, w_mat)
